# Initial kernel scaffold; baseline (speedup 1.0000x reference)
import jax
import jax.numpy as jnp
from jax.experimental import pallas as pl


def kernel(X, pos_enc, Wpos, bpos, Wq, bq, Wk, bk, Wv, bv, Wo, bo, Wpred, bpred, edge_index, graph_ids):
    raise NotImplementedError("write your pallas kernel here")


# speedup vs baseline: 4.9474x; 4.9474x over previous
import functools

import numpy as np
import jax
import jax.numpy as jnp
from jax import lax
from jax.experimental import pallas as pl
from jax.experimental.pallas import tpu as pltpu
from jax.experimental.pallas import tpu_sc as plsc

N = 10000
E = 160000
H = 256
NH = 8
HD = 32
LYR = 8
POS = 8
OUT = 128
G = 64

NPAD = 10240
BN = 256
NB = NPAD // BN

NCORES = 2
NTILES = 16
EPT = E // NTILES
NGRP = EPT // 16
ROWS_PT = NPAD // NTILES

_HEAD_PERM = (np.arange(H) % HD) * NH + (np.arange(H) // HD)


def _proj_first_body(x_ref, w1_ref, b1_ref, w2_ref, b2_ref, *out_refs):
    h = jnp.dot(x_ref[...], w1_ref[...],
                preferred_element_type=jnp.float32) + b1_ref[...]
    for j in range(6):
        out_refs[j][...] = jnp.dot(
            h, w2_ref[:, j * 128:(j + 1) * 128],
            preferred_element_type=jnp.float32) + b2_ref[j * 128:(j + 1) * 128]


def _proj_mid_body(x_ref, wo_ref, bo_ref, w2_ref, b2_ref, *out_refs):
    h = (jnp.dot(x_ref[0], wo_ref[0:128, :], preferred_element_type=jnp.float32)
         + jnp.dot(x_ref[1], wo_ref[128:256, :], preferred_element_type=jnp.float32)
         + bo_ref[...])
    for j in range(6):
        out_refs[j][...] = jnp.dot(
            h, w2_ref[:, j * 128:(j + 1) * 128],
            preferred_element_type=jnp.float32) + b2_ref[j * 128:(j + 1) * 128]


def _proj_last_body(x_ref, wo_ref, bo_ref, h_ref):
    h_ref[...] = (
        jnp.dot(x_ref[0], wo_ref[0:128, :], preferred_element_type=jnp.float32)
        + jnp.dot(x_ref[1], wo_ref[128:256, :], preferred_element_type=jnp.float32)
        + bo_ref[...])


def _pool_body(h_ref, gid_ref, wpred_ref, bpred_ref, out_ref, acc_ref):
    i = pl.program_id(0)

    @pl.when(i == 0)
    def _():
        acc_ref[...] = jnp.zeros_like(acc_ref)

    ids = gid_ref[0]
    onehot = (lax.broadcasted_iota(jnp.int32, (G, BN), 0) == ids
              ).astype(jnp.float32)
    acc_ref[...] += jnp.dot(onehot, h_ref[...],
                            preferred_element_type=jnp.float32)

    @pl.when(i == pl.num_programs(0) - 1)
    def _():
        out_ref[...] = jnp.dot(acc_ref[...], wpred_ref[...],
                               preferred_element_type=jnp.float32) + bpred_ref[...]


def _proj_first(pos_pad, w1, b1, w2, b2):
    outs = [jax.ShapeDtypeStruct((NPAD, 128), jnp.float32)] * 6
    return pl.pallas_call(
        _proj_first_body,
        grid=(NB,),
        in_specs=[
            pl.BlockSpec((BN, POS), lambda i: (i, 0)),
            pl.BlockSpec((POS, H), lambda i: (0, 0)),
            pl.BlockSpec((H,), lambda i: (0,)),
            pl.BlockSpec((H, 3 * H), lambda i: (0, 0)),
            pl.BlockSpec((3 * H,), lambda i: (0,)),
        ],
        out_specs=[pl.BlockSpec((BN, 128), lambda i: (i, 0))] * 6,
        out_shape=outs,
    )(pos_pad, w1, b1, w2, b2)


def _proj_mid(sc_u, wo, bo, w2, b2):
    outs = [jax.ShapeDtypeStruct((NPAD, 128), jnp.float32)] * 6
    return pl.pallas_call(
        _proj_mid_body,
        grid=(NB,),
        in_specs=[
            pl.BlockSpec((2, BN, 128), lambda i: (0, i, 0)),
            pl.BlockSpec((H, H), lambda i: (0, 0)),
            pl.BlockSpec((H,), lambda i: (0,)),
            pl.BlockSpec((H, 3 * H), lambda i: (0, 0)),
            pl.BlockSpec((3 * H,), lambda i: (0,)),
        ],
        out_specs=[pl.BlockSpec((BN, 128), lambda i: (i, 0))] * 6,
        out_shape=outs,
    )(sc_u, wo, bo, w2, b2)


def _proj_last(sc_u, wo, bo):
    return pl.pallas_call(
        _proj_last_body,
        grid=(NB,),
        in_specs=[
            pl.BlockSpec((2, BN, 128), lambda i: (0, i, 0)),
            pl.BlockSpec((H, H), lambda i: (0, 0)),
            pl.BlockSpec((H,), lambda i: (0,)),
        ],
        out_specs=pl.BlockSpec((BN, H), lambda i: (i, 0)),
        out_shape=jax.ShapeDtypeStruct((NPAD, H), jnp.float32),
    )(sc_u, wo, bo)


def _pool_pred(h_final, gid3d, wpred, bpred):
    return pl.pallas_call(
        _pool_body,
        grid=(NB,),
        in_specs=[
            pl.BlockSpec((BN, H), lambda i: (i, 0)),
            pl.BlockSpec((1, 1, BN), lambda i: (i, 0, 0)),
            pl.BlockSpec((H, OUT), lambda i: (0, 0)),
            pl.BlockSpec((OUT,), lambda i: (0,)),
        ],
        out_specs=pl.BlockSpec((G, OUT), lambda i: (0, 0)),
        out_shape=jax.ShapeDtypeStruct((G, OUT), jnp.float32),
        scratch_shapes=[pltpu.VMEM((G, H), jnp.float32)],
        compiler_params=pltpu.CompilerParams(
            dimension_semantics=("arbitrary",)),
    )(h_final, gid3d, wpred, bpred)


def _sc_body(qa, qb, ka, kb, va, vb, rowr, colr, u_ref,
             idx_r, idx_c, qbuf, kbuf, vbuf, wbuf, exsrc, zbuf, dnbuf,
             denom_sh, out_sh, sem):
    c = lax.axis_index("c")
    s = lax.axis_index("s")
    lane = lax.iota(jnp.int32, 16)
    zeros16 = jnp.zeros((16,), jnp.float32)
    ebase = s * EPT
    rbase = s * ROWS_PT
    drows_pt = (NPAD // 32) // NTILES

    pltpu.sync_copy(rowr.at[pl.ds(ebase, EPT)], idx_r)
    pltpu.sync_copy(colr.at[pl.ds(ebase, EPT)], idx_c)

    for i in range(16):
        for j in range(8):
            zbuf[i, pl.ds(j * 16, 16)] = zeros16
            exsrc[i, pl.ds(j * 16, 16)] = zeros16
    for i in range(ROWS_PT // 16):
        pltpu.sync_copy(zbuf, out_sh.at[pl.ds(rbase + i * 16, 16)])
    pltpu.sync_copy(zbuf, denom_sh.at[pl.ds(s * drows_pt, 16)])
    pltpu.sync_copy(zbuf.at[pl.ds(0, drows_pt - 16)],
                    denom_sh.at[pl.ds(s * drows_pt + 16, drows_pt - 16)])
    plsc.subcore_barrier()

    @pl.loop(0, NGRP)
    def _edges(g):
        eb = g * 16
        ridx = idx_r[pl.ds(eb, 16)]
        cidx = idx_c[pl.ds(eb, 16)]

        @pl.when(c == 0)
        def _():
            pltpu.async_copy(qa.at[ridx], qbuf, sem).wait()
            pltpu.async_copy(ka.at[cidx], kbuf, sem).wait()
            pltpu.async_copy(va.at[cidx], vbuf, sem).wait()

        @pl.when(c == 1)
        def _():
            pltpu.async_copy(qb.at[ridx], qbuf, sem).wait()
            pltpu.async_copy(kb.at[cidx], kbuf, sem).wait()
            pltpu.async_copy(vb.at[cidx], vbuf, sem).wait()

        dcol = (ridx % 32) * 4
        for h in range(4):
            acc = jnp.zeros((16,), jnp.float32)
            for j in range(HD):
                col = jnp.full((16,), h * HD + j, jnp.int32)
                qv = plsc.load_gather(qbuf, [lane, col])
                kv = plsc.load_gather(kbuf, [lane, col])
                acc = acc + qv * kv
            ex = jnp.exp(acc)
            plsc.store_scatter(exsrc, [lane, dcol + h], ex)
            for j in range(HD):
                col = jnp.full((16,), h * HD + j, jnp.int32)
                vv = plsc.load_gather(vbuf, [lane, col])
                plsc.store_scatter(wbuf, [lane, col], vv * ex)
        pltpu.sync_copy(exsrc, denom_sh.at[ridx // 32], add=True)
        pltpu.sync_copy(wbuf, out_sh.at[ridx], add=True)
        for h in range(4):
            plsc.store_scatter(exsrc, [lane, dcol + h], zeros16)

    plsc.subcore_barrier()

    pltpu.sync_copy(denom_sh.at[pl.ds(s * drows_pt, drows_pt)], dnbuf)

    @pl.loop(0, ROWS_PT // 16)
    def _writeout(ch):
        pltpu.sync_copy(out_sh.at[pl.ds(rbase + ch * 16, 16)], wbuf)
        node_l = ch * 16 + lane
        drow = node_l // 32
        dcol = (node_l % 32) * 4
        for h in range(4):
            dv = plsc.load_gather(dnbuf, [drow, dcol + h])
            rec = 1.0 / (dv + 1e-9)
            for j in range(HD):
                col = jnp.full((16,), h * HD + j, jnp.int32)
                vv = plsc.load_gather(wbuf, [lane, col])
                plsc.store_scatter(wbuf, [lane, col], vv * rec)

        @pl.when(c == 0)
        def _():
            pltpu.sync_copy(wbuf, u_ref.at[0, pl.ds(rbase + ch * 16, 16)])

        @pl.when(c == 1)
        def _():
            pltpu.sync_copy(wbuf, u_ref.at[1, pl.ds(rbase + ch * 16, 16)])


_sc_sparse = functools.partial(
    pl.kernel,
    out_type=jax.ShapeDtypeStruct((2, NPAD, 128), jnp.float32),
    mesh=plsc.VectorSubcoreMesh(core_axis_name="c", subcore_axis_name="s"),
    scratch_types=[
        pltpu.VMEM((EPT,), jnp.int32),
        pltpu.VMEM((EPT,), jnp.int32),
        pltpu.VMEM((16, 128), jnp.float32),
        pltpu.VMEM((16, 128), jnp.float32),
        pltpu.VMEM((16, 128), jnp.float32),
        pltpu.VMEM((16, 128), jnp.float32),
        pltpu.VMEM((16, 128), jnp.float32),
        pltpu.VMEM((16, 128), jnp.float32),
        pltpu.VMEM(((NPAD // 32) // NTILES, 128), jnp.float32),
        pltpu.VMEM_SHARED((NPAD // 32, 128), jnp.float32),
        pltpu.VMEM_SHARED((NPAD, 128), jnp.float32),
        pltpu.SemaphoreType.DMA,
    ],
    compiler_params=pltpu.CompilerParams(needs_layout_passes=False),
)(_sc_body)


def kernel(X, pos_enc, Wpos, bpos, Wq, bq, Wk, bk, Wv, bv, Wo, bo,
           Wpred, bpred, edge_index, graph_ids):
    del X
    scaling = float(HD) ** -0.5
    perm = _HEAD_PERM

    wq = Wq[:, :, perm] * scaling
    bq_ = bq[:, perm] * scaling
    wk = Wk[:, :, perm]
    bk_ = bk[:, perm]
    wv = Wv[:, :, perm]
    bv_ = bv[:, perm]
    wo = Wo[:, perm, :]

    wqkv = jnp.concatenate([wq, wk, wv], axis=2)
    bqkv = jnp.concatenate([bq_, bk_, bv_], axis=1)

    pos_pad = jnp.zeros((NPAD, POS), jnp.float32).at[:N].set(pos_enc)
    gid3d = jnp.full((NPAD,), G, jnp.int32).at[:N].set(graph_ids)
    gid3d = gid3d.reshape(NB, 1, BN)
    row = edge_index[0]
    col = edge_index[1]

    qkv = _proj_first(pos_pad, Wpos, bpos, wqkv[0], bqkv[0])
    for l in range(LYR):
        sc_u = _sc_sparse(*qkv, row, col)
        if l < LYR - 1:
            qkv = _proj_mid(sc_u, wo[l], bo[l], wqkv[l + 1], bqkv[l + 1])
    h_final = _proj_last(sc_u, wo[LYR - 1], bo[LYR - 1])
    return _pool_pred(h_final, gid3d, Wpred, bpred)
